# Initial kernel scaffold; baseline (speedup 1.0000x reference)
import jax
import jax.numpy as jnp
from jax.experimental import pallas as pl


def kernel(x, edge_index, edge_label_index, Wl1, bl1, Wr1, Wl2, bl2, Wr2, Wd1, bd1, Wd2, bd2):
    raise NotImplementedError("write your pallas kernel here")


# speedup vs baseline: 2.4312x; 2.4312x over previous
import dataclasses
import functools

import jax
import jax.numpy as jnp
from jax import lax
from jax.experimental import pallas as pl
from jax.experimental.pallas import tpu as pltpu
from jax.experimental.pallas import tpu_sc as plsc

F32 = jnp.float32
I32 = jnp.int32

N_NODES = 10000
NP = 10240
TRASH = 10100
E = 320000
EB = 5
EP = 32 * EB * 2048
L = 100000
LB = 25
LP = 32 * LB * 128

_MESH = plsc.VectorSubcoreMesh(core_axis_name="c", subcore_axis_name="s")

_CP = pltpu.CompilerParams()
if "needs_layout_passes" in pltpu.CompilerParams.__dataclass_fields__:
    _CP = dataclasses.replace(_CP, needs_layout_passes=False)


def _seg_kernel(with_counts):
    out_types = [jax.ShapeDtypeStruct((2, NP, 128), F32)]
    scratch = [
        pltpu.VMEM_SHARED((NP, 128), F32),
        pltpu.VMEM((16, 128), I32),
        pltpu.VMEM((16, 128), I32),
        pltpu.VMEM((128, 128), F32),
    ]
    if with_counts:
        out_types.append(jax.ShapeDtypeStruct((2, NP // 128, 128), F32))
        scratch += [
            pltpu.VMEM_SHARED((NP // 128, 128), F32),
            pltpu.VMEM((NP // 128, 128), F32),
            pltpu.VMEM((NP // 128,), I32),
        ]

    def body(y_hbm, src_hbm, dst_hbm, *refs):
        if with_counts:
            (out_hbm, cnt_hbm, acc, src_v, dst_v, rows_v,
             acc_c, hist, io_v) = refs
        else:
            out_hbm, acc, src_v, dst_v, rows_v = refs
        c = lax.axis_index("c")
        s = lax.axis_index("s")
        w = c * 16 + s
        nrow = NP // 128

        @pl.loop(0, 128)
        def _(i):
            @pl.loop(0, 8)
            def _(j):
                rows_v[i, pl.ds(j * 16, 16)] = jnp.zeros((16,), F32)

        @pl.loop(0, 5)
        def _(k):
            pltpu.sync_copy(rows_v, acc.at[pl.ds(s * 640 + k * 128, 128)])
        if with_counts:
            @pl.loop(0, nrow)
            def _(i):
                @pl.loop(0, 8)
                def _(j):
                    hist[i, pl.ds(j * 16, 16)] = jnp.zeros((16,), F32)

            @pl.loop(0, nrow // 16)
            def _(k):
                io_v[pl.ds(k * 16, 16)] = lax.iota(I32, 16) + k * 16
            pltpu.sync_copy(rows_v.at[pl.ds(0, 5)], acc_c.at[pl.ds(s * 5, 5)])
        plsc.subcore_barrier()

        @pl.loop(0, EB)
        def _(b):
            pltpu.sync_copy(src_hbm.at[w * EB + b], src_v)
            pltpu.sync_copy(dst_hbm.at[w * EB + b], dst_v)
            if with_counts:
                @pl.loop(0, 16)
                def _(i):
                    @pl.loop(0, 8)
                    def _(j):
                        idx = dst_v[i, pl.ds(j * 16, 16)]
                        hi = jax.lax.shift_right_logical(idx, 7)
                        lo = jax.lax.bitwise_and(idx, 127)
                        plsc.addupdate_scatter(hist, [hi, lo],
                                               jnp.ones((16,), F32))

            @pl.loop(0, 16)
            def _(i):
                pltpu.sync_copy(y_hbm.at[src_v.at[i]], rows_v)
                pltpu.sync_copy(rows_v, acc.at[dst_v.at[i]], add=True)
        if with_counts:
            pltpu.sync_copy(hist, acc_c.at[io_v], add=True)
        plsc.subcore_barrier()

        pltpu.sync_copy(acc.at[pl.ds(s * 640, 640)],
                        out_hbm.at[c, pl.ds(s * 640, 640)])
        if with_counts:
            @pl.when(s < nrow // 8)
            def _():
                pltpu.sync_copy(acc_c.at[pl.ds(s * 8, 8)],
                                cnt_hbm.at[c, pl.ds(s * 8, 8)])

    return pl.kernel(body, out_type=tuple(out_types), mesh=_MESH,
                     scratch_types=scratch, compiler_params=_CP)


@functools.partial(
    pl.kernel,
    out_type=(jax.ShapeDtypeStruct((LP, 128), F32),
              jax.ShapeDtypeStruct((LP, 128), F32)),
    mesh=_MESH,
    compiler_params=_CP,
    scratch_types=[
        pltpu.VMEM((LB * 128,), I32),
        pltpu.VMEM((LB * 128,), I32),
        pltpu.VMEM((128, 128), F32),
        pltpu.VMEM((128, 128), F32),
    ],
)
def _decoder_gather(p_hbm, q_hbm, sidx_hbm, didx_hbm, ps_hbm, qd_hbm,
                    sidx_v, didx_v, prow, qrow):
    c = lax.axis_index("c")
    s = lax.axis_index("s")
    w = c * 16 + s
    pltpu.sync_copy(sidx_hbm.at[pl.ds(w * LB * 128, LB * 128)], sidx_v)
    pltpu.sync_copy(didx_hbm.at[pl.ds(w * LB * 128, LB * 128)], didx_v)

    @pl.loop(0, LB)
    def _(i):
        pltpu.sync_copy(p_hbm.at[sidx_v.at[pl.ds(i * 128, 128)]], prow)
        pltpu.sync_copy(prow, ps_hbm.at[pl.ds(w * LB * 128 + i * 128, 128)])
        pltpu.sync_copy(q_hbm.at[didx_v.at[pl.ds(i * 128, 128)]], qrow)
        pltpu.sync_copy(qrow, qd_hbm.at[pl.ds(w * LB * 128 + i * 128, 128)])


_W_SPEC = pl.BlockSpec((128, 128), lambda i: (0, 0))
_B_SPEC = pl.BlockSpec((1, 128), lambda i: (0, 0))
_ROW_SPEC = pl.BlockSpec((1024, 128), lambda i: (i, 0))
_SUM_SPEC = pl.BlockSpec((2, 1024, 128), lambda i: (0, i, 0))
_CNT_SPEC = pl.BlockSpec((2, 1024, 1), lambda i: (0, i, 0))


def _tc_a_body(x_ref, wl_ref, wr_ref, bl_ref, y_ref, r_ref):
    xx = x_ref[...]
    y_ref[...] = jnp.dot(xx, wl_ref[...], preferred_element_type=F32)
    r_ref[...] = jnp.dot(xx, wr_ref[...], preferred_element_type=F32) \
        + bl_ref[...]


_tc_a = pl.pallas_call(
    _tc_a_body,
    grid=(NP // 1024,),
    in_specs=[_ROW_SPEC, _W_SPEC, _W_SPEC, _B_SPEC],
    out_specs=[_ROW_SPEC, _ROW_SPEC],
    out_shape=[jax.ShapeDtypeStruct((NP, 128), F32)] * 2,
)


def _tc_mid_body(relu, s_ref, cnt_ref, r_ref, wl_ref, wr_ref, bl_ref,
                 y_ref, r2_ref):
    sm = s_ref[0] + s_ref[1]
    cr = cnt_ref[0] + cnt_ref[1]
    inv = 1.0 / jnp.maximum(cr, 1.0)
    h = sm * inv + r_ref[...]
    if relu:
        h = jnp.maximum(h, 0.0)
    y_ref[...] = jnp.dot(h, wl_ref[...], preferred_element_type=F32)
    r2_ref[...] = jnp.dot(h, wr_ref[...], preferred_element_type=F32) \
        + bl_ref[...]


def _tc_mid(relu):
    return pl.pallas_call(
        functools.partial(_tc_mid_body, relu),
        grid=(NP // 1024,),
        in_specs=[_SUM_SPEC, _CNT_SPEC, _ROW_SPEC, _W_SPEC, _W_SPEC, _B_SPEC],
        out_specs=[_ROW_SPEC, _ROW_SPEC],
        out_shape=[jax.ShapeDtypeStruct((NP, 128), F32)] * 2,
    )


def _tc_d_body(ps_ref, qd_ref, w_ref, b_ref, o_ref):
    z = jnp.maximum(ps_ref[...] + qd_ref[...], 0.0)
    o_ref[...] = jnp.sum(z * w_ref[...], axis=1, keepdims=True) + b_ref[...]


_tc_d = pl.pallas_call(
    _tc_d_body,
    grid=(L // 2000,),
    in_specs=[pl.BlockSpec((2000, 128), lambda i: (i, 0)),
              pl.BlockSpec((2000, 128), lambda i: (i, 0)),
              _B_SPEC, pl.BlockSpec((1, 1), lambda i: (0, 0))],
    out_specs=pl.BlockSpec((2000, 1), lambda i: (i, 0)),
    out_shape=jax.ShapeDtypeStruct((L, 1), F32),
)


def _pad_idx(idx, total, fill, shape):
    idx = idx.astype(I32)
    pad = total - idx.shape[0]
    idx = jnp.concatenate([idx, jnp.full((pad,), fill, I32)])
    return idx.reshape(shape)


def kernel(x, edge_index, edge_label_index,
           Wl1, bl1, Wr1, Wl2, bl2, Wr2, Wd1, bd1, Wd2, bd2):
    xp = jnp.concatenate([x, jnp.zeros((NP - N_NODES, 128), F32)])
    src = _pad_idx(edge_index[0], EP, 0, (32 * EB, 16, 128))
    dst = _pad_idx(edge_index[1], EP, TRASH, (32 * EB, 16, 128))
    lsrc = _pad_idx(edge_label_index[0], LP, 0, (LP,))
    ldst = _pad_idx(edge_label_index[1], LP, 0, (LP,))
    bl1r = bl1.reshape(1, 128)
    bl2r = bl2.reshape(1, 128)
    bd1r = bd1.reshape(1, 128)
    wd2r = Wd2.reshape(1, 128)
    bd2r = bd2.reshape(1, 1)

    y1, r1 = _tc_a(xp, Wl1, Wr1, bl1r)
    s1, cnt_lin = _seg_kernel(True)(y1, src, dst)
    cnt = cnt_lin.reshape(2, NP, 1)
    y2, r2 = _tc_mid(True)(s1, cnt, r1, Wl2, Wr2, bl2r)
    (s2,) = _seg_kernel(False)(y2, src, dst)
    p, q = _tc_mid(False)(s2, cnt, r2, Wd1[:128], Wd1[128:], bd1r)
    ps, qd = _decoder_gather(p, q, lsrc, ldst)
    return _tc_d(ps, qd, wd2r, bd2r)


# speedup vs baseline: 3.2351x; 1.3307x over previous
import dataclasses
import functools

import jax
import jax.numpy as jnp
from jax import lax
from jax.experimental import pallas as pl
from jax.experimental.pallas import tpu as pltpu
from jax.experimental.pallas import tpu_sc as plsc

F32 = jnp.float32
I32 = jnp.int32

N_NODES = 10000
NP = 10240
TRASH = 10100
E = 320000
EB = 5
EP = 32 * EB * 2048
L = 100000
LB = 25
LP = 32 * LB * 128

_MESH = plsc.VectorSubcoreMesh(core_axis_name="c", subcore_axis_name="s")

_CP = pltpu.CompilerParams()
if "needs_layout_passes" in pltpu.CompilerParams.__dataclass_fields__:
    _CP = dataclasses.replace(_CP, needs_layout_passes=False)


@functools.partial(
    pl.kernel,
    out_type=jax.ShapeDtypeStruct((2, NP // 128, 128), F32),
    mesh=_MESH,
    compiler_params=_CP,
    scratch_types=[
        pltpu.VMEM_SHARED((NP // 128, 128), F32),
        pltpu.VMEM((16, 128), I32),
        pltpu.VMEM((NP // 128, 128), F32),
        pltpu.VMEM((NP // 128,), I32),
    ],
)
def _count_kernel(dst_hbm, cnt_hbm, acc_c, dst_v, hist, io_v):
    c = lax.axis_index("c")
    s = lax.axis_index("s")
    w = c * 16 + s
    nrow = NP // 128

    @pl.loop(0, nrow)
    def _(i):
        @pl.loop(0, 8)
        def _(j):
            hist[i, pl.ds(j * 16, 16)] = jnp.zeros((16,), F32)

    @pl.loop(0, nrow // 16)
    def _(k):
        io_v[pl.ds(k * 16, 16)] = lax.iota(I32, 16) + k * 16
    pltpu.sync_copy(hist.at[pl.ds(0, 5)], acc_c.at[pl.ds(s * 5, 5)])
    plsc.subcore_barrier()

    @pl.loop(0, EB)
    def _(b):
        pltpu.sync_copy(dst_hbm.at[w * EB + b], dst_v)

        @pl.loop(0, 16)
        def _(i):
            @pl.loop(0, 8)
            def _(j):
                idx = dst_v[i, pl.ds(j * 16, 16)]
                hi = jax.lax.shift_right_logical(idx, 7)
                lo = jax.lax.bitwise_and(idx, 127)
                plsc.addupdate_scatter(hist, [hi, lo], jnp.ones((16,), F32))
    pltpu.sync_copy(hist, acc_c.at[io_v], add=True)
    plsc.subcore_barrier()

    @pl.when(s < nrow // 8)
    def _():
        pltpu.sync_copy(acc_c.at[pl.ds(s * 8, 8)],
                        cnt_hbm.at[c, pl.ds(s * 8, 8)])


@functools.partial(
    pl.kernel,
    out_type=jax.ShapeDtypeStruct((2, NP, 128), F32),
    mesh=_MESH,
    compiler_params=_CP,
    scratch_types=[
        pltpu.VMEM_SHARED((NP, 128), F32),
        pltpu.VMEM((16, 128), I32),
        pltpu.VMEM((16, 128), I32),
        pltpu.VMEM((128, 128), F32),
        pltpu.VMEM((128, 128), F32),
        pltpu.SemaphoreType.DMA,
        pltpu.SemaphoreType.DMA,
    ],
)
def _seg_kernel(y_hbm, src_hbm, dst_hbm, out_hbm,
                acc, src_v, dst_v, rows_a, rows_b, sem_a, sem_b):
    c = lax.axis_index("c")
    s = lax.axis_index("s")
    w = c * 16 + s

    @pl.loop(0, 128)
    def _(i):
        @pl.loop(0, 8)
        def _(j):
            rows_a[i, pl.ds(j * 16, 16)] = jnp.zeros((16,), F32)

    @pl.loop(0, 5)
    def _(k):
        pltpu.sync_copy(rows_a, acc.at[pl.ds(s * 640 + k * 128, 128)])
    plsc.subcore_barrier()

    @pl.loop(0, EB)
    def _(b):
        pltpu.sync_copy(src_hbm.at[w * EB + b], src_v)
        pltpu.sync_copy(dst_hbm.at[w * EB + b], dst_v)
        pltpu.async_copy(y_hbm.at[src_v.at[0]], rows_a, sem_a)

        @pl.loop(0, 8)
        def _(ii):
            i0 = 2 * ii
            pltpu.async_copy(y_hbm.at[src_v.at[i0 + 1]], rows_b, sem_b)
            pltpu.make_async_copy(y_hbm.at[src_v.at[i0]], rows_a,
                                  sem_a).wait()
            pltpu.sync_copy(rows_a, acc.at[dst_v.at[i0]], add=True)

            @pl.when(ii < 7)
            def _():
                pltpu.async_copy(y_hbm.at[src_v.at[i0 + 2]], rows_a, sem_a)
            pltpu.make_async_copy(y_hbm.at[src_v.at[i0 + 1]], rows_b,
                                  sem_b).wait()
            pltpu.sync_copy(rows_b, acc.at[dst_v.at[i0 + 1]], add=True)
    plsc.subcore_barrier()

    pltpu.sync_copy(acc.at[pl.ds(s * 640, 640)],
                    out_hbm.at[c, pl.ds(s * 640, 640)])


@functools.partial(
    pl.kernel,
    out_type=(jax.ShapeDtypeStruct((LP, 128), F32),
              jax.ShapeDtypeStruct((LP, 128), F32)),
    mesh=_MESH,
    compiler_params=_CP,
    scratch_types=[
        pltpu.VMEM((LB * 128,), I32),
        pltpu.VMEM((LB * 128,), I32),
        pltpu.VMEM((128, 128), F32),
        pltpu.VMEM((128, 128), F32),
        pltpu.SemaphoreType.DMA,
        pltpu.SemaphoreType.DMA,
    ],
)
def _decoder_gather(p_hbm, q_hbm, sidx_hbm, didx_hbm, ps_hbm, qd_hbm,
                    sidx_v, didx_v, prow, qrow, semp, semq):
    c = lax.axis_index("c")
    s = lax.axis_index("s")
    w = c * 16 + s
    pltpu.sync_copy(sidx_hbm.at[pl.ds(w * LB * 128, LB * 128)], sidx_v)
    pltpu.sync_copy(didx_hbm.at[pl.ds(w * LB * 128, LB * 128)], didx_v)
    pltpu.async_copy(p_hbm.at[sidx_v.at[pl.ds(0, 128)]], prow, semp)
    pltpu.async_copy(q_hbm.at[didx_v.at[pl.ds(0, 128)]], qrow, semq)

    @pl.loop(0, LB)
    def _(i):
        pltpu.make_async_copy(p_hbm.at[sidx_v.at[pl.ds(i * 128, 128)]],
                              prow, semp).wait()
        pltpu.sync_copy(prow, ps_hbm.at[pl.ds(w * LB * 128 + i * 128, 128)])

        @pl.when(i < LB - 1)
        def _():
            pltpu.async_copy(p_hbm.at[sidx_v.at[pl.ds(i * 128 + 128, 128)]],
                             prow, semp)
        pltpu.make_async_copy(q_hbm.at[didx_v.at[pl.ds(i * 128, 128)]],
                              qrow, semq).wait()
        pltpu.sync_copy(qrow, qd_hbm.at[pl.ds(w * LB * 128 + i * 128, 128)])

        @pl.when(i < LB - 1)
        def _():
            pltpu.async_copy(q_hbm.at[didx_v.at[pl.ds(i * 128 + 128, 128)]],
                             qrow, semq)


_W_SPEC = pl.BlockSpec((128, 128), lambda i: (0, 0))
_B_SPEC = pl.BlockSpec((1, 128), lambda i: (0, 0))
_ROW_SPEC = pl.BlockSpec((1024, 128), lambda i: (i, 0))
_SUM_SPEC = pl.BlockSpec((2, 1024, 128), lambda i: (0, i, 0))
_CNT_SPEC = pl.BlockSpec((2, 1024, 1), lambda i: (0, i, 0))


def _tc_a_body(x_ref, wl_ref, wr_ref, bl_ref, y_ref, r_ref):
    xx = x_ref[...]
    y_ref[...] = jnp.dot(xx, wl_ref[...], preferred_element_type=F32)
    r_ref[...] = jnp.dot(xx, wr_ref[...], preferred_element_type=F32) \
        + bl_ref[...]


_tc_a = pl.pallas_call(
    _tc_a_body,
    grid=(NP // 1024,),
    in_specs=[_ROW_SPEC, _W_SPEC, _W_SPEC, _B_SPEC],
    out_specs=[_ROW_SPEC, _ROW_SPEC],
    out_shape=[jax.ShapeDtypeStruct((NP, 128), F32)] * 2,
)


def _tc_mid_body(relu, s_ref, cnt_ref, r_ref, wl_ref, wr_ref, bl_ref,
                 y_ref, r2_ref):
    sm = s_ref[0] + s_ref[1]
    cr = cnt_ref[0] + cnt_ref[1]
    inv = 1.0 / jnp.maximum(cr, 1.0)
    h = sm * inv + r_ref[...]
    if relu:
        h = jnp.maximum(h, 0.0)
    y_ref[...] = jnp.dot(h, wl_ref[...], preferred_element_type=F32)
    r2_ref[...] = jnp.dot(h, wr_ref[...], preferred_element_type=F32) \
        + bl_ref[...]


def _tc_mid(relu):
    return pl.pallas_call(
        functools.partial(_tc_mid_body, relu),
        grid=(NP // 1024,),
        in_specs=[_SUM_SPEC, _CNT_SPEC, _ROW_SPEC, _W_SPEC, _W_SPEC, _B_SPEC],
        out_specs=[_ROW_SPEC, _ROW_SPEC],
        out_shape=[jax.ShapeDtypeStruct((NP, 128), F32)] * 2,
    )


def _tc_d_body(ps_ref, qd_ref, w_ref, b_ref, o_ref):
    z = jnp.maximum(ps_ref[...] + qd_ref[...], 0.0)
    o_ref[...] = jnp.sum(z * w_ref[...], axis=1, keepdims=True) + b_ref[...]


_tc_d = pl.pallas_call(
    _tc_d_body,
    grid=(L // 2000,),
    in_specs=[pl.BlockSpec((2000, 128), lambda i: (i, 0)),
              pl.BlockSpec((2000, 128), lambda i: (i, 0)),
              _B_SPEC, pl.BlockSpec((1, 1), lambda i: (0, 0))],
    out_specs=pl.BlockSpec((2000, 1), lambda i: (i, 0)),
    out_shape=jax.ShapeDtypeStruct((L, 1), F32),
)


def _pad_idx(idx, total, fill, shape):
    idx = idx.astype(I32)
    pad = total - idx.shape[0]
    idx = jnp.concatenate([idx, jnp.full((pad,), fill, I32)])
    return idx.reshape(shape)


def kernel(x, edge_index, edge_label_index,
           Wl1, bl1, Wr1, Wl2, bl2, Wr2, Wd1, bd1, Wd2, bd2):
    xp = jnp.concatenate([x, jnp.zeros((NP - N_NODES, 128), F32)])
    src = _pad_idx(edge_index[0], EP, 0, (32 * EB, 16, 128))
    dst = _pad_idx(edge_index[1], EP, TRASH, (32 * EB, 16, 128))
    lsrc = _pad_idx(edge_label_index[0], LP, 0, (LP,))
    ldst = _pad_idx(edge_label_index[1], LP, 0, (LP,))
    bl1r = bl1.reshape(1, 128)
    bl2r = bl2.reshape(1, 128)
    bd1r = bd1.reshape(1, 128)
    wd2r = Wd2.reshape(1, 128)
    bd2r = bd2.reshape(1, 1)

    cnt_lin = _count_kernel(dst)
    y1, r1 = _tc_a(xp, Wl1, Wr1, bl1r)
    s1 = _seg_kernel(y1, src, dst)
    cnt = cnt_lin.reshape(2, NP, 1)
    y2, r2 = _tc_mid(True)(s1, cnt, r1, Wl2, Wr2, bl2r)
    s2 = _seg_kernel(y2, src, dst)
    p, q = _tc_mid(False)(s2, cnt, r2, Wd1[:128], Wd1[128:], bd1r)
    ps, qd = _decoder_gather(p, q, lsrc, ldst)
    return _tc_d(ps, qd, wd2r, bd2r)


# speedup vs baseline: 3.4699x; 1.0726x over previous
import dataclasses
import functools

import jax
import jax.numpy as jnp
from jax import lax
from jax.experimental import pallas as pl
from jax.experimental.pallas import tpu as pltpu
from jax.experimental.pallas import tpu_sc as plsc

F32 = jnp.float32
I32 = jnp.int32

N_NODES = 10000
NP = 10240
TRASH = 10100
E = 320000
EB = 5
EP = 32 * EB * 2048
EB0, EB1 = 8, 2
L = 100000
LB = 25
LP = 32 * LB * 128
LB0, LB1 = 40, 10

_MESH = plsc.VectorSubcoreMesh(core_axis_name="c", subcore_axis_name="s")

_CP = pltpu.CompilerParams()
if "needs_layout_passes" in pltpu.CompilerParams.__dataclass_fields__:
    _CP = dataclasses.replace(_CP, needs_layout_passes=False)


@functools.partial(
    pl.kernel,
    out_type=jax.ShapeDtypeStruct((2, NP // 128, 128), F32),
    mesh=_MESH,
    compiler_params=_CP,
    scratch_types=[
        pltpu.VMEM_SHARED((NP // 128, 128), F32),
        pltpu.VMEM((16, 128), I32),
        pltpu.VMEM((NP // 128, 128), F32),
        pltpu.VMEM((NP // 128,), I32),
    ],
)
def _count_kernel(dst_hbm, cnt_hbm, acc_c, dst_v, hist, io_v):
    c = lax.axis_index("c")
    s = lax.axis_index("s")
    w = c * 16 + s
    nrow = NP // 128

    @pl.loop(0, nrow)
    def _(i):
        @pl.loop(0, 8)
        def _(j):
            hist[i, pl.ds(j * 16, 16)] = jnp.zeros((16,), F32)

    @pl.loop(0, nrow // 16)
    def _(k):
        io_v[pl.ds(k * 16, 16)] = lax.iota(I32, 16) + k * 16
    pltpu.sync_copy(hist.at[pl.ds(0, 5)], acc_c.at[pl.ds(s * 5, 5)])
    plsc.subcore_barrier()

    @pl.loop(0, EB)
    def _(b):
        pltpu.sync_copy(dst_hbm.at[w * EB + b], dst_v)

        @pl.loop(0, 16)
        def _(i):
            @pl.loop(0, 8)
            def _(j):
                idx = dst_v[i, pl.ds(j * 16, 16)]
                hi = jax.lax.shift_right_logical(idx, 7)
                lo = jax.lax.bitwise_and(idx, 127)
                plsc.addupdate_scatter(hist, [hi, lo], jnp.ones((16,), F32))
    pltpu.sync_copy(hist, acc_c.at[io_v], add=True)
    plsc.subcore_barrier()

    @pl.when(s < nrow // 8)
    def _():
        pltpu.sync_copy(acc_c.at[pl.ds(s * 8, 8)],
                        cnt_hbm.at[c, pl.ds(s * 8, 8)])


@functools.partial(
    pl.kernel,
    out_type=jax.ShapeDtypeStruct((2, NP, 128), F32),
    mesh=_MESH,
    compiler_params=_CP,
    scratch_types=[
        pltpu.VMEM_SHARED((NP, 128), F32),
        pltpu.VMEM((16, 128), I32),
        pltpu.VMEM((16, 128), I32),
        pltpu.VMEM((128, 128), F32),
        pltpu.VMEM((128, 128), F32),
        pltpu.SemaphoreType.DMA,
        pltpu.SemaphoreType.DMA,
    ],
)
def _seg_kernel(y_hbm, src_hbm, dst_hbm, out_hbm,
                acc, src_v, dst_v, rows_a, rows_b, sem_a, sem_b):
    c = lax.axis_index("c")
    s = lax.axis_index("s")
    w = c * 16 + s

    @pl.loop(0, 128)
    def _(i):
        @pl.loop(0, 8)
        def _(j):
            rows_a[i, pl.ds(j * 16, 16)] = jnp.zeros((16,), F32)

    @pl.loop(0, 5)
    def _(k):
        pltpu.sync_copy(rows_a, acc.at[pl.ds(s * 640 + k * 128, 128)])
    plsc.subcore_barrier()

    def _block(blk):
        pltpu.sync_copy(src_hbm.at[blk], src_v)
        pltpu.sync_copy(dst_hbm.at[blk], dst_v)
        pltpu.async_copy(y_hbm.at[src_v.at[0]], rows_a, sem_a)

        @pl.loop(0, 8)
        def _(ii):
            i0 = 2 * ii
            pltpu.async_copy(y_hbm.at[src_v.at[i0 + 1]], rows_b, sem_b)
            pltpu.make_async_copy(y_hbm.at[src_v.at[i0]], rows_a,
                                  sem_a).wait()
            pltpu.sync_copy(rows_a, acc.at[dst_v.at[i0]], add=True)

            @pl.when(ii < 7)
            def _():
                pltpu.async_copy(y_hbm.at[src_v.at[i0 + 2]], rows_a, sem_a)
            pltpu.make_async_copy(y_hbm.at[src_v.at[i0 + 1]], rows_b,
                                  sem_b).wait()
            pltpu.sync_copy(rows_b, acc.at[dst_v.at[i0 + 1]], add=True)

    @pl.when(c == 0)
    def _():
        @pl.loop(0, EB0)
        def _(b):
            _block(s * EB0 + b)

    @pl.when(c == 1)
    def _():
        @pl.loop(0, EB1)
        def _(b):
            _block(16 * EB0 + s * EB1 + b)
    plsc.subcore_barrier()

    pltpu.sync_copy(acc.at[pl.ds(s * 640, 640)],
                    out_hbm.at[c, pl.ds(s * 640, 640)])


@functools.partial(
    pl.kernel,
    out_type=(jax.ShapeDtypeStruct((LP, 128), F32),
              jax.ShapeDtypeStruct((LP, 128), F32)),
    mesh=_MESH,
    compiler_params=_CP,
    scratch_types=[
        pltpu.VMEM((LB0 * 128,), I32),
        pltpu.VMEM((LB0 * 128,), I32),
        pltpu.VMEM((128, 128), F32),
        pltpu.VMEM((128, 128), F32),
        pltpu.SemaphoreType.DMA,
        pltpu.SemaphoreType.DMA,
    ],
)
def _decoder_gather(p_hbm, q_hbm, sidx_hbm, didx_hbm, ps_hbm, qd_hbm,
                    sidx_v, didx_v, prow, qrow, semp, semq):
    c = lax.axis_index("c")
    s = lax.axis_index("s")

    def _run(nsteps, base):
        n = nsteps * 128
        pltpu.sync_copy(sidx_hbm.at[pl.ds(base, n)],
                        sidx_v.at[pl.ds(0, n)])
        pltpu.sync_copy(didx_hbm.at[pl.ds(base, n)],
                        didx_v.at[pl.ds(0, n)])
        pltpu.async_copy(p_hbm.at[sidx_v.at[pl.ds(0, 128)]], prow, semp)
        pltpu.async_copy(q_hbm.at[didx_v.at[pl.ds(0, 128)]], qrow, semq)

        @pl.loop(0, nsteps)
        def _(i):
            pltpu.make_async_copy(p_hbm.at[sidx_v.at[pl.ds(i * 128, 128)]],
                                  prow, semp).wait()
            pltpu.sync_copy(prow, ps_hbm.at[pl.ds(base + i * 128, 128)])

            @pl.when(i < nsteps - 1)
            def _():
                pltpu.async_copy(
                    p_hbm.at[sidx_v.at[pl.ds(i * 128 + 128, 128)]],
                    prow, semp)
            pltpu.make_async_copy(q_hbm.at[didx_v.at[pl.ds(i * 128, 128)]],
                                  qrow, semq).wait()
            pltpu.sync_copy(qrow, qd_hbm.at[pl.ds(base + i * 128, 128)])

            @pl.when(i < nsteps - 1)
            def _():
                pltpu.async_copy(
                    q_hbm.at[didx_v.at[pl.ds(i * 128 + 128, 128)]],
                    qrow, semq)

    @pl.when(c == 0)
    def _():
        _run(LB0, s * LB0 * 128)

    @pl.when(c == 1)
    def _():
        _run(LB1, 16 * LB0 * 128 + s * LB1 * 128)


_W_SPEC = pl.BlockSpec((128, 128), lambda i: (0, 0))
_B_SPEC = pl.BlockSpec((1, 128), lambda i: (0, 0))
_ROW_SPEC = pl.BlockSpec((1024, 128), lambda i: (i, 0))
_SUM_SPEC = pl.BlockSpec((2, 1024, 128), lambda i: (0, i, 0))
_CNT_SPEC = pl.BlockSpec((2, 1024, 1), lambda i: (0, i, 0))


def _tc_a_body(x_ref, wl_ref, wr_ref, bl_ref, y_ref, r_ref):
    xx = x_ref[...]
    y_ref[...] = jnp.dot(xx, wl_ref[...], preferred_element_type=F32)
    r_ref[...] = jnp.dot(xx, wr_ref[...], preferred_element_type=F32) \
        + bl_ref[...]


_tc_a = pl.pallas_call(
    _tc_a_body,
    grid=(NP // 1024,),
    in_specs=[_ROW_SPEC, _W_SPEC, _W_SPEC, _B_SPEC],
    out_specs=[_ROW_SPEC, _ROW_SPEC],
    out_shape=[jax.ShapeDtypeStruct((NP, 128), F32)] * 2,
)


def _tc_mid_body(relu, s_ref, cnt_ref, r_ref, wl_ref, wr_ref, bl_ref,
                 y_ref, r2_ref):
    sm = s_ref[0] + s_ref[1]
    cr = cnt_ref[0] + cnt_ref[1]
    inv = 1.0 / jnp.maximum(cr, 1.0)
    h = sm * inv + r_ref[...]
    if relu:
        h = jnp.maximum(h, 0.0)
    y_ref[...] = jnp.dot(h, wl_ref[...], preferred_element_type=F32)
    r2_ref[...] = jnp.dot(h, wr_ref[...], preferred_element_type=F32) \
        + bl_ref[...]


def _tc_mid(relu):
    return pl.pallas_call(
        functools.partial(_tc_mid_body, relu),
        grid=(NP // 1024,),
        in_specs=[_SUM_SPEC, _CNT_SPEC, _ROW_SPEC, _W_SPEC, _W_SPEC, _B_SPEC],
        out_specs=[_ROW_SPEC, _ROW_SPEC],
        out_shape=[jax.ShapeDtypeStruct((NP, 128), F32)] * 2,
    )


def _tc_d_body(ps_ref, qd_ref, w_ref, b_ref, o_ref):
    z = jnp.maximum(ps_ref[...] + qd_ref[...], 0.0)
    o_ref[...] = jnp.sum(z * w_ref[...], axis=1, keepdims=True) + b_ref[...]


_tc_d = pl.pallas_call(
    _tc_d_body,
    grid=(L // 2000,),
    in_specs=[pl.BlockSpec((2000, 128), lambda i: (i, 0)),
              pl.BlockSpec((2000, 128), lambda i: (i, 0)),
              _B_SPEC, pl.BlockSpec((1, 1), lambda i: (0, 0))],
    out_specs=pl.BlockSpec((2000, 1), lambda i: (i, 0)),
    out_shape=jax.ShapeDtypeStruct((L, 1), F32),
)


def _pad_idx(idx, total, fill, shape):
    idx = idx.astype(I32)
    pad = total - idx.shape[0]
    idx = jnp.concatenate([idx, jnp.full((pad,), fill, I32)])
    return idx.reshape(shape)


def kernel(x, edge_index, edge_label_index,
           Wl1, bl1, Wr1, Wl2, bl2, Wr2, Wd1, bd1, Wd2, bd2):
    xp = jnp.concatenate([x, jnp.zeros((NP - N_NODES, 128), F32)])
    src = _pad_idx(edge_index[0], EP, 0, (32 * EB, 16, 128))
    dst = _pad_idx(edge_index[1], EP, TRASH, (32 * EB, 16, 128))
    lsrc = _pad_idx(edge_label_index[0], LP, 0, (LP,))
    ldst = _pad_idx(edge_label_index[1], LP, 0, (LP,))
    bl1r = bl1.reshape(1, 128)
    bl2r = bl2.reshape(1, 128)
    bd1r = bd1.reshape(1, 128)
    wd2r = Wd2.reshape(1, 128)
    bd2r = bd2.reshape(1, 1)

    cnt_lin = _count_kernel(dst)
    y1, r1 = _tc_a(xp, Wl1, Wr1, bl1r)
    s1 = _seg_kernel(y1, src, dst)
    cnt = cnt_lin.reshape(2, NP, 1)
    y2, r2 = _tc_mid(True)(s1, cnt, r1, Wl2, Wr2, bl2r)
    s2 = _seg_kernel(y2, src, dst)
    p, q = _tc_mid(False)(s2, cnt, r2, Wd1[:128], Wd1[128:], bd1r)
    ps, qd = _decoder_gather(p, q, lsrc, ldst)
    return _tc_d(ps, qd, wd2r, bd2r)
